# Initial kernel scaffold; baseline (speedup 1.0000x reference)
import jax
import jax.numpy as jnp
from jax.experimental import pallas as pl


def kernel(x, edge_index, batch, edge_count, in_degree_inv, out_degree_inv, num_count, sequence, emb_table, weight_in, weight_out, gru_w_ih, gru_w_hh, gru_b_ih, gru_b_hh, W1_w, W1_b, W2_w, W2_b, q_w, q_b, W3_w, W3_b):
    raise NotImplementedError("write your pallas kernel here")


# speedup vs baseline: 2.0743x; 2.0743x over previous
import functools

import jax
import jax.numpy as jnp
from jax import lax
from jax.experimental import pallas as pl
from jax.experimental.pallas import tpu as pltpu
from jax.experimental.pallas import tpu_sc as plsc

N = 50000
H = 64
B = 256
E = 800000
N_NODE = 100000

NC = 2
NS = 16
NW = NC * NS

GP_ROWS = 128
G_BURSTS = 13
ROWS_W = GP_ROWS * G_BURSTS
NP = ROWS_W * NW

EP = 819200
ET = EP // NS
CH = 256
NCH = ET // CH
CR = CH // 128
NHALF = N // 2
ACC_ROWS = 25088
ZR = ACC_ROWS // NS

BLK = 2000
NBLK = N // BLK
NNP = 102400
VBLK = 5120
NVBLK = NNP // VBLK


def _mesh():
    return plsc.VectorSubcoreMesh(core_axis_name="c", subcore_axis_name="s")


def _sc_emb_gather(table, idx2):
    @functools.partial(
        pl.kernel,
        out_type=jax.ShapeDtypeStruct((NP, H), jnp.float32),
        mesh=_mesh(),
        compiler_params=pltpu.CompilerParams(use_tc_tiling_on_sc=False),
        scratch_types=[
            pltpu.VMEM((G_BURSTS, GP_ROWS), jnp.int32),
            pltpu.VMEM((ROWS_W, H), jnp.float32),
            pltpu.SemaphoreType.DMA,
        ],
    )
    def k(table_h, idx_h, out_h, idx_v, rows_v, sem):
        w = lax.axis_index("s") * NC + lax.axis_index("c")
        pltpu.sync_copy(idx_h.at[w], idx_v)
        for r in range(G_BURSTS):
            def sub1(i, _, r=r):
                sl = pl.ds(i * 16, 16)
                idx_v[r, sl] = idx_v[r, sl] - 1
                return 0
            lax.fori_loop(0, GP_ROWS // 16, sub1, 0)
        cps = [
            pltpu.async_copy(table_h.at[idx_v.at[r]],
                             rows_v.at[pl.ds(r * GP_ROWS, GP_ROWS)], sem)
            for r in range(G_BURSTS)
        ]
        for cp in cps:
            cp.wait()
        pltpu.sync_copy(rows_v, out_h.at[pl.ds(w * ROWS_W, ROWS_W)])

    return k(table, idx2)


def _sc_vn_gather(hidden, last_idx):
    rw = B // NW

    @functools.partial(
        pl.kernel,
        out_type=jax.ShapeDtypeStruct((B, H), jnp.float32),
        mesh=_mesh(),
        compiler_params=pltpu.CompilerParams(use_tc_tiling_on_sc=False),
        scratch_types=[
            pltpu.VMEM((rw,), jnp.int32),
            pltpu.VMEM((rw, H), jnp.float32),
            pltpu.SemaphoreType.DMA,
        ],
    )
    def k(hid_h, idx_h, out_h, idx_v, rows_v, sem):
        w = lax.axis_index("s") * NC + lax.axis_index("c")
        pltpu.sync_copy(idx_h.at[pl.ds(w * rw, rw)], idx_v)
        pltpu.async_copy(hid_h.at[idx_v], rows_v, sem).wait()
        pltpu.sync_copy(rows_v, out_h.at[pl.ds(w * rw, rw)])

    return k(hidden, last_idx)


def _sc_edge_agg(m1, m2, src2, dst2, ec2, din2, dout2):
    out_t = (jax.ShapeDtypeStruct((N, H), jnp.float32),
             jax.ShapeDtypeStruct((N, H), jnp.float32))

    @functools.partial(
        pl.kernel,
        out_type=out_t,
        mesh=_mesh(),
        compiler_params=pltpu.CompilerParams(use_tc_tiling_on_sc=False),
        scratch_types=[
            pltpu.VMEM_SHARED((ACC_ROWS, H), jnp.float32),
            pltpu.VMEM((CR, 128), jnp.int32),
            pltpu.VMEM((CR, 128), jnp.int32),
            pltpu.VMEM((CR, 128), jnp.int32),
            pltpu.VMEM((CR, 128), jnp.float32),
            pltpu.VMEM((CR, 128), jnp.float32),
            pltpu.VMEM((CH, H), jnp.float32),
            pltpu.SemaphoreType.DMA,
        ],
    )
    def k(m1_h, m2_h, src_h, dst_h, ec_h, din_h, dout_h, agg1_h, agg2_h,
          acc, gidx_v, tidx_v, adj_v, ec_v, ew_v, rows_v, sem):
        c = lax.axis_index("c")
        s = lax.axis_index("s")
        half0 = c * NHALF

        for d in range(2):
            mat_h = m1_h if d == 0 else m2_h
            g_h = src_h if d == 0 else dst_h
            t_h = dst_h if d == 0 else src_h
            w_h = din_h if d == 0 else dout_h
            out_h = agg1_h if d == 0 else agg2_h

            def zrow(kk, _):
                zz = jnp.zeros((16,), jnp.float32)
                for q in range(H // 16):
                    rows_v[kk, pl.ds(q * 16, 16)] = zz
                return 0
            lax.fori_loop(0, CH, zrow, 0)
            for off in range(0, ZR - CH + 1, CH):
                pltpu.sync_copy(rows_v, acc.at[pl.ds(s * ZR + off, CH)])
            rem = ZR % CH
            if rem:
                pltpu.sync_copy(rows_v.at[pl.ds(0, rem)],
                                acc.at[pl.ds(s * ZR + ZR - rem, rem)])
            plsc.subcore_barrier()

            def chunk(j, _):
                rb = s * (ET // 128) + j * CR
                pltpu.sync_copy(g_h.at[pl.ds(rb, CR)], gidx_v)
                pltpu.sync_copy(t_h.at[pl.ds(rb, CR)], tidx_v)
                pltpu.sync_copy(ec_h.at[pl.ds(rb, CR)], ec_v)
                pltpu.sync_copy(w_h.at[pl.ds(rb, CR)], ew_v)
                cps = [
                    pltpu.async_copy(mat_h.at[gidx_v.at[r]],
                                     rows_v.at[pl.ds(r * 128, 128)], sem)
                    for r in range(CR)
                ]
                def prep_r(r, _):
                    def prep_i(i, _):
                        sl = pl.ds(i * 16, 16)
                        ew_v[r, sl] = ec_v[r, sl] * ew_v[r, sl]
                        t = tidx_v[r, sl] - half0
                        ok = (t >= 0) & (t < NHALF)
                        adj_v[r, sl] = jnp.where(ok, t, NHALF)
                        return 0
                    lax.fori_loop(0, 8, prep_i, 0)
                    return 0
                lax.fori_loop(0, CR, prep_r, 0)
                for cp in cps:
                    cp.wait()

                def scale_r(r, _):
                    def scale_b(kb, _):
                        w16 = ew_v[r, pl.ds(kb * 16, 16)]
                        base = r * 128 + kb * 16
                        for kk in range(16):
                            wk = w16[kk]
                            for q in range(H // 16):
                                sl = pl.ds(q * 16, 16)
                                rows_v[base + kk, sl] = rows_v[base + kk, sl] * wk
                        return 0
                    lax.fori_loop(0, 8, scale_b, 0)
                    return 0
                lax.fori_loop(0, CR, scale_r, 0)

                for r in range(CR):
                    pltpu.sync_copy(rows_v.at[pl.ds(r * 128, 128)],
                                    acc.at[adj_v.at[r]], add=True)
                return 0
            lax.fori_loop(0, NCH, chunk, 0)
            plsc.subcore_barrier()

            @pl.when(s < NS - 1)
            def _():
                pltpu.sync_copy(acc.at[pl.ds(s * 1568, 1568)],
                                out_h.at[pl.ds(half0 + s * 1568, 1568)])

            @pl.when(s == NS - 1)
            def _():
                pltpu.sync_copy(acc.at[pl.ds((NS - 1) * 1568, 1480)],
                                out_h.at[pl.ds(half0 + (NS - 1) * 1568, 1480)])
            plsc.subcore_barrier()

    return k(m1, m2, src2, dst2, ec2, din2, dout2)


def _tc_m1m2(h, win, wout):
    def body(h_ref, wi_ref, wo_ref, m1_ref, m2_ref):
        hh = h_ref[...]
        m1_ref[...] = jnp.dot(hh, wi_ref[...], preferred_element_type=jnp.float32)
        m2_ref[...] = jnp.dot(hh, wo_ref[...], preferred_element_type=jnp.float32)

    return pl.pallas_call(
        body,
        grid=(NBLK,),
        in_specs=[
            pl.BlockSpec((BLK, H), lambda i: (i, 0)),
            pl.BlockSpec((H, H), lambda i: (0, 0)),
            pl.BlockSpec((H, H), lambda i: (0, 0)),
        ],
        out_specs=[pl.BlockSpec((BLK, H), lambda i: (i, 0))] * 2,
        out_shape=[jax.ShapeDtypeStruct((N, H), jnp.float32)] * 2,
    )(h, win, wout)


def _tc_gru(agg1, agg2, h, wihT, whhT, bih, bhh):
    def body(a1_ref, a2_ref, h_ref, wih_ref, whh_ref, bih_ref, bhh_ref, out_ref):
        a1 = a1_ref[...]
        a2 = a2_ref[...]
        hh = h_ref[...]
        wih = wih_ref[...]
        gi = (jnp.dot(a1, wih[:H], preferred_element_type=jnp.float32)
              + jnp.dot(a2, wih[H:], preferred_element_type=jnp.float32)
              + bih_ref[...])
        gh = jnp.dot(hh, whh_ref[...], preferred_element_type=jnp.float32) + bhh_ref[...]
        r = jax.nn.sigmoid(gi[:, :H] + gh[:, :H])
        z = jax.nn.sigmoid(gi[:, H:2 * H] + gh[:, H:2 * H])
        ng = jnp.tanh(gi[:, 2 * H:] + r * gh[:, 2 * H:])
        out_ref[...] = (1.0 - z) * ng + z * hh

    return pl.pallas_call(
        body,
        grid=(NBLK,),
        in_specs=[
            pl.BlockSpec((BLK, H), lambda i: (i, 0)),
            pl.BlockSpec((BLK, H), lambda i: (i, 0)),
            pl.BlockSpec((BLK, H), lambda i: (i, 0)),
            pl.BlockSpec((2 * H, 3 * H), lambda i: (0, 0)),
            pl.BlockSpec((H, 3 * H), lambda i: (0, 0)),
            pl.BlockSpec((1, 3 * H), lambda i: (0, 0)),
            pl.BlockSpec((1, 3 * H), lambda i: (0, 0)),
        ],
        out_specs=pl.BlockSpec((BLK, H), lambda i: (i, 0)),
        out_shape=jax.ShapeDtypeStruct((N, H), jnp.float32),
    )(agg1, agg2, h, wihT, whhT, bih, bhh)


def _tc_last_idx(batch3):
    def body(b_ref, cnt_ref, last_ref):
        i = pl.program_id(0)

        @pl.when(i == 0)
        def _():
            cnt_ref[...] = jnp.zeros_like(cnt_ref)
            last_ref[...] = jnp.zeros_like(last_ref)

        bcol = b_ref[0, 0, :].reshape(BLK, 1)
        iota_b = lax.broadcasted_iota(jnp.int32, (BLK, B), 1)
        oh = jnp.where(bcol == iota_b, 1.0, 0.0)
        cnt_ref[...] += jnp.sum(oh, axis=0, keepdims=True)

        @pl.when(i == NBLK - 1)
        def _():
            cnt = cnt_ref[...]
            ir = lax.broadcasted_iota(jnp.int32, (B, B), 0)
            ic = lax.broadcasted_iota(jnp.int32, (B, B), 1)
            tri = jnp.where(ir <= ic, 1.0, 0.0)
            csum = jnp.dot(cnt, tri, preferred_element_type=jnp.float32)
            ci = csum.astype(jnp.int32)
            last_ref[...] = jnp.where(cnt > 0.0, ci - 1, 0)

    return pl.pallas_call(
        body,
        grid=(NBLK,),
        in_specs=[pl.BlockSpec((1, 1, BLK), lambda i: (i, 0, 0))],
        out_specs=[pl.BlockSpec((1, B), lambda i: (0, 0)),
                   pl.BlockSpec((1, B), lambda i: (0, 0))],
        out_shape=[jax.ShapeDtypeStruct((1, B), jnp.float32),
                   jax.ShapeDtypeStruct((1, B), jnp.int32)],
    )(batch3)


def _tc_alpha_sg(hidden, batch3, nc3, v_n, w1T, w2T, b12, qw, qb):
    def body(h_ref, b_ref, n_ref, vn_ref, w1_ref, w2_ref, b12_ref, qw_ref,
             qb_ref, sg_ref):
        i = pl.program_id(0)

        @pl.when(i == 0)
        def _():
            sg_ref[...] = jnp.zeros_like(sg_ref)

        hh = h_ref[...]
        bcol = b_ref[0, 0, :].reshape(BLK, 1)
        iota_b = lax.broadcasted_iota(jnp.int32, (BLK, B), 1)
        oh = jnp.where(bcol == iota_b, 1.0, 0.0)
        vrep = jnp.dot(oh, vn_ref[...], preferred_element_type=jnp.float32)
        t = jax.nn.sigmoid(
            jnp.dot(vrep, w1_ref[...], preferred_element_type=jnp.float32)
            + jnp.dot(hh, w2_ref[...], preferred_element_type=jnp.float32)
            + b12_ref[...])
        alpha = jnp.sum(t * qw_ref[...], axis=1, keepdims=True) + qb_ref[...]
        s = n_ref[0, 0, :].reshape(BLK, 1) * alpha * hh
        sg_ref[...] += lax.dot_general(oh, s, (((0,), (0,)), ((), ())),
                                       preferred_element_type=jnp.float32)

    return pl.pallas_call(
        body,
        grid=(NBLK,),
        in_specs=[
            pl.BlockSpec((BLK, H), lambda i: (i, 0)),
            pl.BlockSpec((1, 1, BLK), lambda i: (i, 0, 0)),
            pl.BlockSpec((1, 1, BLK), lambda i: (i, 0, 0)),
            pl.BlockSpec((B, H), lambda i: (0, 0)),
            pl.BlockSpec((H, H), lambda i: (0, 0)),
            pl.BlockSpec((H, H), lambda i: (0, 0)),
            pl.BlockSpec((1, H), lambda i: (0, 0)),
            pl.BlockSpec((1, H), lambda i: (0, 0)),
            pl.BlockSpec((1, 1), lambda i: (0, 0)),
        ],
        out_specs=pl.BlockSpec((B, H), lambda i: (0, 0)),
        out_shape=jax.ShapeDtypeStruct((B, H), jnp.float32),
    )(hidden, batch3, nc3, v_n, w1T, w2T, b12, qw, qb)


def _tc_logits(v_n, s_g, w3T, b3, emb):
    def body(vn_ref, sg_ref, w3_ref, b3_ref, emb_ref, out_ref):
        w3 = w3_ref[...]
        sh = (jnp.dot(vn_ref[...], w3[:H], preferred_element_type=jnp.float32)
              + jnp.dot(sg_ref[...], w3[H:], preferred_element_type=jnp.float32)
              + b3_ref[...])
        out_ref[...] = lax.dot_general(sh, emb_ref[...],
                                       (((1,), (1,)), ((), ())),
                                       preferred_element_type=jnp.float32)

    return pl.pallas_call(
        body,
        grid=(NVBLK,),
        in_specs=[
            pl.BlockSpec((B, H), lambda i: (0, 0)),
            pl.BlockSpec((B, H), lambda i: (0, 0)),
            pl.BlockSpec((2 * H, H), lambda i: (0, 0)),
            pl.BlockSpec((1, H), lambda i: (0, 0)),
            pl.BlockSpec((VBLK, H), lambda i: (i, 0)),
        ],
        out_specs=pl.BlockSpec((B, VBLK), lambda i: (0, i)),
        out_shape=jax.ShapeDtypeStruct((B, NNP), jnp.float32),
    )(v_n, s_g, w3T, b3, emb)


def kernel(x, edge_index, batch, edge_count, in_degree_inv, out_degree_inv,
           num_count, sequence, emb_table, weight_in, weight_out, gru_w_ih,
           gru_w_hh, gru_b_ih, gru_b_hh, W1_w, W1_b, W2_w, W2_b, q_w, q_b,
           W3_w, W3_b):
    xi = x.reshape(-1).astype(jnp.int32)
    xp2 = jnp.concatenate([xi, jnp.ones((NP - N,), jnp.int32)]).reshape(
        NW, G_BURSTS, 128)
    h = _sc_emb_gather(emb_table, xp2)[:N]

    m1, m2 = _tc_m1m2(h, weight_in[0], weight_out[0])

    pad = EP - E
    zi = jnp.zeros((pad,), jnp.int32)
    zf = jnp.zeros((pad,), jnp.float32)
    src2 = jnp.concatenate([edge_index[0].astype(jnp.int32), zi]).reshape(-1, 128)
    dst2 = jnp.concatenate([edge_index[1].astype(jnp.int32), zi]).reshape(-1, 128)
    ec2 = jnp.concatenate([edge_count, zf]).reshape(-1, 128)
    din2 = jnp.concatenate([in_degree_inv, zf]).reshape(-1, 128)
    dout2 = jnp.concatenate([out_degree_inv, zf]).reshape(-1, 128)
    agg1, agg2 = _sc_edge_agg(m1, m2, src2, dst2, ec2, din2, dout2)

    hidden = _tc_gru(agg1, agg2, h, gru_w_ih.T, gru_w_hh.T,
                     gru_b_ih.reshape(1, -1), gru_b_hh.reshape(1, -1))

    batch3 = batch.astype(jnp.int32).reshape(NBLK, 1, BLK)
    _, last_idx = _tc_last_idx(batch3)
    v_n = _sc_vn_gather(hidden, last_idx.reshape(-1))

    nc3 = num_count.reshape(NBLK, 1, BLK)
    s_g = _tc_alpha_sg(hidden, batch3, nc3, v_n, W1_w.T, W2_w.T,
                       (W1_b + W2_b).reshape(1, -1), q_w.reshape(1, -1),
                       q_b.reshape(1, 1))

    embp = jnp.pad(emb_table, ((0, NNP - N_NODE), (0, 0)))
    z = _tc_logits(v_n, s_g, W3_w.T, W3_b.reshape(1, -1), embp)
    return z[:, :N_NODE]


# speedup vs baseline: 2.1389x; 1.0312x over previous
import functools

import jax
import jax.numpy as jnp
from jax import lax
from jax.experimental import pallas as pl
from jax.experimental.pallas import tpu as pltpu
from jax.experimental.pallas import tpu_sc as plsc

N = 50000
H = 64
B = 256
E = 800000
N_NODE = 100000

NC = 2
NS = 16
NW = NC * NS

GP_ROWS = 128
G_BURSTS = 13
ROWS_W = GP_ROWS * G_BURSTS
NP = ROWS_W * NW

EP = 819200
ET = EP // NS
CH = 128
SB = 8
NSUP = ET // (CH * SB)
NHALF = N // 2
ACC_ROWS = 25088
ZR = ACC_ROWS // NS

BLK = 2000
NBLK = N // BLK
NNP = 102400
VBLK = 5120
NVBLK = NNP // VBLK


def _mesh():
    return plsc.VectorSubcoreMesh(core_axis_name="c", subcore_axis_name="s")


def _sc_emb_gather(table, idx2):
    @functools.partial(
        pl.kernel,
        out_type=jax.ShapeDtypeStruct((NP, H), jnp.float32),
        mesh=_mesh(),
        compiler_params=pltpu.CompilerParams(use_tc_tiling_on_sc=False),
        scratch_types=[
            pltpu.VMEM((G_BURSTS, GP_ROWS), jnp.int32),
            pltpu.VMEM((ROWS_W, H), jnp.float32),
            pltpu.SemaphoreType.DMA,
        ],
    )
    def k(table_h, idx_h, out_h, idx_v, rows_v, sem):
        w = lax.axis_index("s") * NC + lax.axis_index("c")
        pltpu.sync_copy(idx_h.at[w], idx_v)
        for r in range(G_BURSTS):
            def sub1(i, _, r=r):
                sl = pl.ds(i * 16, 16)
                idx_v[r, sl] = idx_v[r, sl] - 1
                return 0
            lax.fori_loop(0, GP_ROWS // 16, sub1, 0)
        cps = [
            pltpu.async_copy(table_h.at[idx_v.at[r]],
                             rows_v.at[pl.ds(r * GP_ROWS, GP_ROWS)], sem)
            for r in range(G_BURSTS)
        ]
        for cp in cps:
            cp.wait()
        pltpu.sync_copy(rows_v, out_h.at[pl.ds(w * ROWS_W, ROWS_W)])

    return k(table, idx2)


def _sc_vn_gather(hidden, last_idx):
    rw = B // NW

    @functools.partial(
        pl.kernel,
        out_type=jax.ShapeDtypeStruct((B, H), jnp.float32),
        mesh=_mesh(),
        compiler_params=pltpu.CompilerParams(use_tc_tiling_on_sc=False),
        scratch_types=[
            pltpu.VMEM((rw,), jnp.int32),
            pltpu.VMEM((rw, H), jnp.float32),
            pltpu.SemaphoreType.DMA,
        ],
    )
    def k(hid_h, idx_h, out_h, idx_v, rows_v, sem):
        w = lax.axis_index("s") * NC + lax.axis_index("c")
        pltpu.sync_copy(idx_h.at[pl.ds(w * rw, rw)], idx_v)
        pltpu.async_copy(hid_h.at[idx_v], rows_v, sem).wait()
        pltpu.sync_copy(rows_v, out_h.at[pl.ds(w * rw, rw)])

    return k(hidden, last_idx)


def _sc_edge_agg(m1, m2, src2, dst2, ec2, din2, dout2):
    out_t = (jax.ShapeDtypeStruct((N, H), jnp.float32),
             jax.ShapeDtypeStruct((N, H), jnp.float32))

    @functools.partial(
        pl.kernel,
        out_type=out_t,
        mesh=_mesh(),
        compiler_params=pltpu.CompilerParams(use_tc_tiling_on_sc=False),
        scratch_types=[
            pltpu.VMEM_SHARED((ACC_ROWS, H), jnp.float32),
            pltpu.VMEM((SB, 128), jnp.int32),
            pltpu.VMEM((SB, 128), jnp.int32),
            pltpu.VMEM((SB, 128), jnp.int32),
            pltpu.VMEM((SB, 128), jnp.float32),
            pltpu.VMEM((SB, 128), jnp.float32),
            pltpu.VMEM((2 * CH, H), jnp.float32),
            pltpu.SemaphoreType.DMA,
            pltpu.SemaphoreType.DMA,
        ],
    )
    def k(m1_h, m2_h, src_h, dst_h, ec_h, din_h, dout_h, agg1_h, agg2_h,
          acc, gidx_v, tidx_v, adj_v, ec_v, ew_v, rows_v, semi, semg):
        c = lax.axis_index("c")
        s = lax.axis_index("s")
        half0 = c * NHALF

        for d in range(2):
            mat_h = m1_h if d == 0 else m2_h
            g_h = src_h if d == 0 else dst_h
            t_h = dst_h if d == 0 else src_h
            w_h = din_h if d == 0 else dout_h
            out_h = agg1_h if d == 0 else agg2_h

            def zrow(kk, _):
                zz = jnp.zeros((16,), jnp.float32)
                for q in range(H // 16):
                    rows_v[kk, pl.ds(q * 16, 16)] = zz
                return 0
            lax.fori_loop(0, 2 * CH, zrow, 0)
            for off in range(0, ZR - 2 * CH + 1, 2 * CH):
                pltpu.sync_copy(rows_v, acc.at[pl.ds(s * ZR + off, 2 * CH)])
            rem = ZR % (2 * CH)
            if rem:
                pltpu.sync_copy(rows_v.at[pl.ds(0, rem)],
                                acc.at[pl.ds(s * ZR + ZR - rem, rem)])
            plsc.subcore_barrier()

            def super_chunk(sj, _):
                rb = s * (ET // 128) + sj * SB
                cp_i = [pltpu.async_copy(g_h.at[pl.ds(rb, SB)], gidx_v, semi),
                        pltpu.async_copy(t_h.at[pl.ds(rb, SB)], tidx_v, semi),
                        pltpu.async_copy(ec_h.at[pl.ds(rb, SB)], ec_v, semi),
                        pltpu.async_copy(w_h.at[pl.ds(rb, SB)], ew_v, semi)]
                for cp in cp_i:
                    cp.wait()

                def prep_r(r, _):
                    def prep_i(i, _):
                        sl = pl.ds(i * 16, 16)
                        ew_v[r, sl] = ec_v[r, sl] * ew_v[r, sl]
                        t = tidx_v[r, sl] - half0
                        ok = (t >= 0) & (t < NHALF)
                        adj_v[r, sl] = jnp.where(ok, t, NHALF)
                        return 0
                    lax.fori_loop(0, 8, prep_i, 0)
                    return 0
                lax.fori_loop(0, SB, prep_r, 0)

                cps = [None, None]
                cps[0] = pltpu.async_copy(mat_h.at[gidx_v.at[0]],
                                          rows_v.at[pl.ds(0, CH)], semg)
                for kc in range(SB):
                    b = kc % 2
                    if kc + 1 < SB:
                        cps[1 - b] = pltpu.async_copy(
                            mat_h.at[gidx_v.at[kc + 1]],
                            rows_v.at[pl.ds((1 - b) * CH, CH)], semg)
                    cps[b].wait()

                    def scale_b(kb, _, kc=kc, b=b):
                        w16 = ew_v[kc, pl.ds(kb * 16, 16)]
                        base = b * CH + kb * 16
                        for kk in range(16):
                            wk = w16[kk]
                            for q in range(H // 16):
                                sl = pl.ds(q * 16, 16)
                                rows_v[base + kk, sl] = rows_v[base + kk, sl] * wk
                        return 0
                    lax.fori_loop(0, CH // 16, scale_b, 0)

                    pltpu.sync_copy(rows_v.at[pl.ds(b * CH, CH)],
                                    acc.at[adj_v.at[kc]], add=True)
                return 0
            lax.fori_loop(0, NSUP, super_chunk, 0)
            plsc.subcore_barrier()

            @pl.when(s < NS - 1)
            def _():
                pltpu.sync_copy(acc.at[pl.ds(s * 1568, 1568)],
                                out_h.at[pl.ds(half0 + s * 1568, 1568)])

            @pl.when(s == NS - 1)
            def _():
                pltpu.sync_copy(acc.at[pl.ds((NS - 1) * 1568, 1480)],
                                out_h.at[pl.ds(half0 + (NS - 1) * 1568, 1480)])
            plsc.subcore_barrier()

    return k(m1, m2, src2, dst2, ec2, din2, dout2)


def _tc_m1m2(h, win, wout):
    def body(h_ref, wi_ref, wo_ref, m1_ref, m2_ref):
        hh = h_ref[...]
        m1_ref[...] = jnp.dot(hh, wi_ref[...], preferred_element_type=jnp.float32)
        m2_ref[...] = jnp.dot(hh, wo_ref[...], preferred_element_type=jnp.float32)

    return pl.pallas_call(
        body,
        grid=(NBLK,),
        in_specs=[
            pl.BlockSpec((BLK, H), lambda i: (i, 0)),
            pl.BlockSpec((H, H), lambda i: (0, 0)),
            pl.BlockSpec((H, H), lambda i: (0, 0)),
        ],
        out_specs=[pl.BlockSpec((BLK, H), lambda i: (i, 0))] * 2,
        out_shape=[jax.ShapeDtypeStruct((N, H), jnp.float32)] * 2,
    )(h, win, wout)


def _tc_gru(agg1, agg2, h, wihT, whhT, bih, bhh):
    def body(a1_ref, a2_ref, h_ref, wih_ref, whh_ref, bih_ref, bhh_ref, out_ref):
        a1 = a1_ref[...]
        a2 = a2_ref[...]
        hh = h_ref[...]
        wih = wih_ref[...]
        gi = (jnp.dot(a1, wih[:H], preferred_element_type=jnp.float32)
              + jnp.dot(a2, wih[H:], preferred_element_type=jnp.float32)
              + bih_ref[...])
        gh = jnp.dot(hh, whh_ref[...], preferred_element_type=jnp.float32) + bhh_ref[...]
        r = jax.nn.sigmoid(gi[:, :H] + gh[:, :H])
        z = jax.nn.sigmoid(gi[:, H:2 * H] + gh[:, H:2 * H])
        ng = jnp.tanh(gi[:, 2 * H:] + r * gh[:, 2 * H:])
        out_ref[...] = (1.0 - z) * ng + z * hh

    return pl.pallas_call(
        body,
        grid=(NBLK,),
        in_specs=[
            pl.BlockSpec((BLK, H), lambda i: (i, 0)),
            pl.BlockSpec((BLK, H), lambda i: (i, 0)),
            pl.BlockSpec((BLK, H), lambda i: (i, 0)),
            pl.BlockSpec((2 * H, 3 * H), lambda i: (0, 0)),
            pl.BlockSpec((H, 3 * H), lambda i: (0, 0)),
            pl.BlockSpec((1, 3 * H), lambda i: (0, 0)),
            pl.BlockSpec((1, 3 * H), lambda i: (0, 0)),
        ],
        out_specs=pl.BlockSpec((BLK, H), lambda i: (i, 0)),
        out_shape=jax.ShapeDtypeStruct((N, H), jnp.float32),
    )(agg1, agg2, h, wihT, whhT, bih, bhh)


def _tc_last_idx(batch3):
    def body(b_ref, cnt_ref, last_ref):
        i = pl.program_id(0)

        @pl.when(i == 0)
        def _():
            cnt_ref[...] = jnp.zeros_like(cnt_ref)
            last_ref[...] = jnp.zeros_like(last_ref)

        bcol = b_ref[0, 0, :].reshape(BLK, 1)
        iota_b = lax.broadcasted_iota(jnp.int32, (BLK, B), 1)
        oh = jnp.where(bcol == iota_b, 1.0, 0.0)
        cnt_ref[...] += jnp.sum(oh, axis=0, keepdims=True)

        @pl.when(i == NBLK - 1)
        def _():
            cnt = cnt_ref[...]
            ir = lax.broadcasted_iota(jnp.int32, (B, B), 0)
            ic = lax.broadcasted_iota(jnp.int32, (B, B), 1)
            tri = jnp.where(ir <= ic, 1.0, 0.0)
            csum = jnp.dot(cnt, tri, preferred_element_type=jnp.float32)
            ci = csum.astype(jnp.int32)
            last_ref[...] = jnp.where(cnt > 0.0, ci - 1, 0)

    return pl.pallas_call(
        body,
        grid=(NBLK,),
        in_specs=[pl.BlockSpec((1, 1, BLK), lambda i: (i, 0, 0))],
        out_specs=[pl.BlockSpec((1, B), lambda i: (0, 0)),
                   pl.BlockSpec((1, B), lambda i: (0, 0))],
        out_shape=[jax.ShapeDtypeStruct((1, B), jnp.float32),
                   jax.ShapeDtypeStruct((1, B), jnp.int32)],
    )(batch3)


def _tc_alpha_sg(hidden, batch3, nc3, v_n, w1T, w2T, b12, qw, qb):
    def body(h_ref, b_ref, n_ref, vn_ref, w1_ref, w2_ref, b12_ref, qw_ref,
             qb_ref, sg_ref):
        i = pl.program_id(0)

        @pl.when(i == 0)
        def _():
            sg_ref[...] = jnp.zeros_like(sg_ref)

        hh = h_ref[...]
        bcol = b_ref[0, 0, :].reshape(BLK, 1)
        iota_b = lax.broadcasted_iota(jnp.int32, (BLK, B), 1)
        oh = jnp.where(bcol == iota_b, 1.0, 0.0)
        vrep = jnp.dot(oh, vn_ref[...], preferred_element_type=jnp.float32)
        t = jax.nn.sigmoid(
            jnp.dot(vrep, w1_ref[...], preferred_element_type=jnp.float32)
            + jnp.dot(hh, w2_ref[...], preferred_element_type=jnp.float32)
            + b12_ref[...])
        alpha = jnp.sum(t * qw_ref[...], axis=1, keepdims=True) + qb_ref[...]
        s = n_ref[0, 0, :].reshape(BLK, 1) * alpha * hh
        sg_ref[...] += lax.dot_general(oh, s, (((0,), (0,)), ((), ())),
                                       preferred_element_type=jnp.float32)

    return pl.pallas_call(
        body,
        grid=(NBLK,),
        in_specs=[
            pl.BlockSpec((BLK, H), lambda i: (i, 0)),
            pl.BlockSpec((1, 1, BLK), lambda i: (i, 0, 0)),
            pl.BlockSpec((1, 1, BLK), lambda i: (i, 0, 0)),
            pl.BlockSpec((B, H), lambda i: (0, 0)),
            pl.BlockSpec((H, H), lambda i: (0, 0)),
            pl.BlockSpec((H, H), lambda i: (0, 0)),
            pl.BlockSpec((1, H), lambda i: (0, 0)),
            pl.BlockSpec((1, H), lambda i: (0, 0)),
            pl.BlockSpec((1, 1), lambda i: (0, 0)),
        ],
        out_specs=pl.BlockSpec((B, H), lambda i: (0, 0)),
        out_shape=jax.ShapeDtypeStruct((B, H), jnp.float32),
    )(hidden, batch3, nc3, v_n, w1T, w2T, b12, qw, qb)


def _tc_logits(v_n, s_g, w3T, b3, emb):
    def body(vn_ref, sg_ref, w3_ref, b3_ref, emb_ref, out_ref):
        w3 = w3_ref[...]
        sh = (jnp.dot(vn_ref[...], w3[:H], preferred_element_type=jnp.float32)
              + jnp.dot(sg_ref[...], w3[H:], preferred_element_type=jnp.float32)
              + b3_ref[...])
        out_ref[...] = lax.dot_general(sh, emb_ref[...],
                                       (((1,), (1,)), ((), ())),
                                       preferred_element_type=jnp.float32)

    return pl.pallas_call(
        body,
        grid=(NVBLK,),
        in_specs=[
            pl.BlockSpec((B, H), lambda i: (0, 0)),
            pl.BlockSpec((B, H), lambda i: (0, 0)),
            pl.BlockSpec((2 * H, H), lambda i: (0, 0)),
            pl.BlockSpec((1, H), lambda i: (0, 0)),
            pl.BlockSpec((VBLK, H), lambda i: (i, 0)),
        ],
        out_specs=pl.BlockSpec((B, VBLK), lambda i: (0, i)),
        out_shape=jax.ShapeDtypeStruct((B, NNP), jnp.float32),
    )(v_n, s_g, w3T, b3, emb)


def kernel(x, edge_index, batch, edge_count, in_degree_inv, out_degree_inv,
           num_count, sequence, emb_table, weight_in, weight_out, gru_w_ih,
           gru_w_hh, gru_b_ih, gru_b_hh, W1_w, W1_b, W2_w, W2_b, q_w, q_b,
           W3_w, W3_b):
    xi = x.reshape(-1).astype(jnp.int32)
    xp2 = jnp.concatenate([xi, jnp.ones((NP - N,), jnp.int32)]).reshape(
        NW, G_BURSTS, 128)
    h = _sc_emb_gather(emb_table, xp2)[:N]

    m1, m2 = _tc_m1m2(h, weight_in[0], weight_out[0])

    pad = EP - E
    zi = jnp.zeros((pad,), jnp.int32)
    zf = jnp.zeros((pad,), jnp.float32)
    src2 = jnp.concatenate([edge_index[0].astype(jnp.int32), zi]).reshape(-1, 128)
    dst2 = jnp.concatenate([edge_index[1].astype(jnp.int32), zi]).reshape(-1, 128)
    ec2 = jnp.concatenate([edge_count, zf]).reshape(-1, 128)
    din2 = jnp.concatenate([in_degree_inv, zf]).reshape(-1, 128)
    dout2 = jnp.concatenate([out_degree_inv, zf]).reshape(-1, 128)
    agg1, agg2 = _sc_edge_agg(m1, m2, src2, dst2, ec2, din2, dout2)

    hidden = _tc_gru(agg1, agg2, h, gru_w_ih.T, gru_w_hh.T,
                     gru_b_ih.reshape(1, -1), gru_b_hh.reshape(1, -1))

    batch3 = batch.astype(jnp.int32).reshape(NBLK, 1, BLK)
    _, last_idx = _tc_last_idx(batch3)
    v_n = _sc_vn_gather(hidden, last_idx.reshape(-1))

    nc3 = num_count.reshape(NBLK, 1, BLK)
    s_g = _tc_alpha_sg(hidden, batch3, nc3, v_n, W1_w.T, W2_w.T,
                       (W1_b + W2_b).reshape(1, -1), q_w.reshape(1, -1),
                       q_b.reshape(1, 1))

    embp = jnp.pad(emb_table, ((0, NNP - N_NODE), (0, 0)))
    z = _tc_logits(v_n, s_g, W3_w.T, W3_b.reshape(1, -1), embp)
    return z[:, :N_NODE]


# speedup vs baseline: 4.3784x; 2.0470x over previous
import functools

import jax
import jax.numpy as jnp
from jax import lax
from jax.experimental import pallas as pl
from jax.experimental.pallas import tpu as pltpu
from jax.experimental.pallas import tpu_sc as plsc

N = 50000
H = 64
B = 256
E = 800000
N_NODE = 100000

NC = 2
NS = 16
NW = NC * NS

GP_ROWS = 128
G_BURSTS = 13
ROWS_W = GP_ROWS * G_BURSTS
NP = ROWS_W * NW

EP = 819200
ET = EP // NS
CH = 128
SB = 8
RING = 4
NSUP = ET // (CH * SB)
HH = H // 2
ZR = N // NS

BLK = 2000
NBLK = N // BLK
NNP = 102400
VBLK = 5120
NVBLK = NNP // VBLK


def _mesh():
    return plsc.VectorSubcoreMesh(core_axis_name="c", subcore_axis_name="s")


def _sc_emb_gather(table, idx2):
    @functools.partial(
        pl.kernel,
        out_type=jax.ShapeDtypeStruct((NP, H), jnp.float32),
        mesh=_mesh(),
        compiler_params=pltpu.CompilerParams(use_tc_tiling_on_sc=False),
        scratch_types=[
            pltpu.VMEM((G_BURSTS, GP_ROWS), jnp.int32),
            pltpu.VMEM((ROWS_W, H), jnp.float32),
            pltpu.SemaphoreType.DMA,
        ],
    )
    def k(table_h, idx_h, out_h, idx_v, rows_v, sem):
        w = lax.axis_index("s") * NC + lax.axis_index("c")
        pltpu.sync_copy(idx_h.at[w], idx_v)
        for r in range(G_BURSTS):
            def sub1(i, _, r=r):
                sl = pl.ds(i * 16, 16)
                idx_v[r, sl] = idx_v[r, sl] - 1
                return 0
            lax.fori_loop(0, GP_ROWS // 16, sub1, 0)
        cps = [
            pltpu.async_copy(table_h.at[idx_v.at[r]],
                             rows_v.at[pl.ds(r * GP_ROWS, GP_ROWS)], sem)
            for r in range(G_BURSTS)
        ]
        for cp in cps:
            cp.wait()
        pltpu.sync_copy(rows_v, out_h.at[pl.ds(w * ROWS_W, ROWS_W)])

    return k(table, idx2)


def _sc_vn_gather(hidden, last_idx):
    rw = B // NW

    @functools.partial(
        pl.kernel,
        out_type=jax.ShapeDtypeStruct((B, H), jnp.float32),
        mesh=_mesh(),
        compiler_params=pltpu.CompilerParams(use_tc_tiling_on_sc=False),
        scratch_types=[
            pltpu.VMEM((rw,), jnp.int32),
            pltpu.VMEM((rw, H), jnp.float32),
            pltpu.SemaphoreType.DMA,
        ],
    )
    def k(hid_h, idx_h, out_h, idx_v, rows_v, sem):
        w = lax.axis_index("s") * NC + lax.axis_index("c")
        pltpu.sync_copy(idx_h.at[pl.ds(w * rw, rw)], idx_v)
        pltpu.async_copy(hid_h.at[idx_v], rows_v, sem).wait()
        pltpu.sync_copy(rows_v, out_h.at[pl.ds(w * rw, rw)])

    return k(hidden, last_idx)


def _sc_edge_agg(m1a, m1b, m2a, m2b, src2, dst2, ec2, din2, dout2):
    out_t = tuple(jax.ShapeDtypeStruct((N, HH), jnp.float32) for _ in range(4))

    @functools.partial(
        pl.kernel,
        out_type=out_t,
        mesh=_mesh(),
        compiler_params=pltpu.CompilerParams(use_tc_tiling_on_sc=False),
        scratch_types=[
            pltpu.VMEM_SHARED((N, HH), jnp.float32),
            pltpu.VMEM((SB, 128), jnp.int32),
            pltpu.VMEM((SB, 128), jnp.int32),
            pltpu.VMEM((SB, 128), jnp.float32),
            pltpu.VMEM((SB, 128), jnp.float32),
            pltpu.VMEM((RING * CH, HH), jnp.float32),
            pltpu.SemaphoreType.DMA,
            pltpu.SemaphoreType.DMA,
        ],
    )
    def k(m1a_h, m1b_h, m2a_h, m2b_h, src_h, dst_h, ec_h, din_h, dout_h,
          o1a_h, o1b_h, o2a_h, o2b_h,
          acc, gidx_v, tidx_v, ec_v, ew_v, rows_v, semi, semg):
        c = lax.axis_index("c")
        s = lax.axis_index("s")

        def one_direction(mat_h, g_h, t_h, w_h, out_h):
            def zrow(kk, _):
                zz = jnp.zeros((16,), jnp.float32)
                for q in range(HH // 16):
                    rows_v[kk, pl.ds(q * 16, 16)] = zz
                return 0
            lax.fori_loop(0, RING * CH, zrow, 0)
            zb = RING * CH
            for off in range(0, ZR - zb + 1, zb):
                pltpu.sync_copy(rows_v, acc.at[pl.ds(s * ZR + off, zb)])
            rem = ZR % zb
            if rem:
                pltpu.sync_copy(rows_v.at[pl.ds(0, rem)],
                                acc.at[pl.ds(s * ZR + ZR - rem, rem)])
            plsc.subcore_barrier()

            def super_chunk(sj, _):
                rb = s * (ET // 128) + sj * SB
                cp_i = [pltpu.async_copy(g_h.at[pl.ds(rb, SB)], gidx_v, semi),
                        pltpu.async_copy(t_h.at[pl.ds(rb, SB)], tidx_v, semi),
                        pltpu.async_copy(ec_h.at[pl.ds(rb, SB)], ec_v, semi),
                        pltpu.async_copy(w_h.at[pl.ds(rb, SB)], ew_v, semi)]
                for cp in cp_i:
                    cp.wait()

                def prep_r(r, _):
                    def prep_i(i, _):
                        sl = pl.ds(i * 16, 16)
                        ew_v[r, sl] = ec_v[r, sl] * ew_v[r, sl]
                        return 0
                    lax.fori_loop(0, 8, prep_i, 0)
                    return 0
                lax.fori_loop(0, SB, prep_r, 0)

                cps = [None] * RING
                for p in range(RING - 1):
                    cps[p] = pltpu.async_copy(
                        mat_h.at[gidx_v.at[p]],
                        rows_v.at[pl.ds(p * CH, CH)], semg)
                for kc in range(SB):
                    b = kc % RING
                    nx = kc + RING - 1
                    if nx < SB:
                        cps[nx % RING] = pltpu.async_copy(
                            mat_h.at[gidx_v.at[nx]],
                            rows_v.at[pl.ds((nx % RING) * CH, CH)], semg)
                    cps[b].wait()

                    def scale_b(kb, _, kc=kc, b=b):
                        w16 = ew_v[kc, pl.ds(kb * 16, 16)]
                        base = b * CH + kb * 16
                        for kk in range(16):
                            wk = w16[kk]
                            for q in range(HH // 16):
                                sl = pl.ds(q * 16, 16)
                                rows_v[base + kk, sl] = rows_v[base + kk, sl] * wk
                        return 0
                    lax.fori_loop(0, CH // 16, scale_b, 0)

                    pltpu.sync_copy(rows_v.at[pl.ds(b * CH, CH)],
                                    acc.at[tidx_v.at[kc]], add=True)
                return 0
            lax.fori_loop(0, NSUP, super_chunk, 0)
            plsc.subcore_barrier()

            pltpu.sync_copy(acc.at[pl.ds(s * ZR, ZR)],
                            out_h.at[pl.ds(s * ZR, ZR)])
            plsc.subcore_barrier()

        @pl.when(c == 0)
        def _():
            one_direction(m1a_h, src_h, dst_h, din_h, o1a_h)
            one_direction(m2a_h, dst_h, src_h, dout_h, o2a_h)

        @pl.when(c == 1)
        def _():
            one_direction(m1b_h, src_h, dst_h, din_h, o1b_h)
            one_direction(m2b_h, dst_h, src_h, dout_h, o2b_h)

    return k(m1a, m1b, m2a, m2b, src2, dst2, ec2, din2, dout2)


def _tc_m1m2(h, win, wout):
    def body(h_ref, wi_ref, wo_ref, m1a_ref, m1b_ref, m2a_ref, m2b_ref):
        hh = h_ref[...]
        m1 = jnp.dot(hh, wi_ref[...], preferred_element_type=jnp.float32)
        m2 = jnp.dot(hh, wo_ref[...], preferred_element_type=jnp.float32)
        m1a_ref[...] = m1[:, :HH]
        m1b_ref[...] = m1[:, HH:]
        m2a_ref[...] = m2[:, :HH]
        m2b_ref[...] = m2[:, HH:]

    return pl.pallas_call(
        body,
        grid=(NBLK,),
        in_specs=[
            pl.BlockSpec((BLK, H), lambda i: (i, 0)),
            pl.BlockSpec((H, H), lambda i: (0, 0)),
            pl.BlockSpec((H, H), lambda i: (0, 0)),
        ],
        out_specs=[pl.BlockSpec((BLK, HH), lambda i: (i, 0))] * 4,
        out_shape=[jax.ShapeDtypeStruct((N, HH), jnp.float32)] * 4,
    )(h, win, wout)


def _tc_gru(a1a, a1b, a2a, a2b, h, wihT, whhT, bih, bhh):
    def body(a1a_ref, a1b_ref, a2a_ref, a2b_ref, h_ref, wih_ref, whh_ref,
             bih_ref, bhh_ref, out_ref):
        hh = h_ref[...]
        wih = wih_ref[...]
        gi = (jnp.dot(a1a_ref[...], wih[:HH], preferred_element_type=jnp.float32)
              + jnp.dot(a1b_ref[...], wih[HH:H], preferred_element_type=jnp.float32)
              + jnp.dot(a2a_ref[...], wih[H:H + HH], preferred_element_type=jnp.float32)
              + jnp.dot(a2b_ref[...], wih[H + HH:], preferred_element_type=jnp.float32)
              + bih_ref[...])
        gh = jnp.dot(hh, whh_ref[...], preferred_element_type=jnp.float32) + bhh_ref[...]
        r = jax.nn.sigmoid(gi[:, :H] + gh[:, :H])
        z = jax.nn.sigmoid(gi[:, H:2 * H] + gh[:, H:2 * H])
        ng = jnp.tanh(gi[:, 2 * H:] + r * gh[:, 2 * H:])
        out_ref[...] = (1.0 - z) * ng + z * hh

    return pl.pallas_call(
        body,
        grid=(NBLK,),
        in_specs=[
            pl.BlockSpec((BLK, HH), lambda i: (i, 0)),
            pl.BlockSpec((BLK, HH), lambda i: (i, 0)),
            pl.BlockSpec((BLK, HH), lambda i: (i, 0)),
            pl.BlockSpec((BLK, HH), lambda i: (i, 0)),
            pl.BlockSpec((BLK, H), lambda i: (i, 0)),
            pl.BlockSpec((2 * H, 3 * H), lambda i: (0, 0)),
            pl.BlockSpec((H, 3 * H), lambda i: (0, 0)),
            pl.BlockSpec((1, 3 * H), lambda i: (0, 0)),
            pl.BlockSpec((1, 3 * H), lambda i: (0, 0)),
        ],
        out_specs=pl.BlockSpec((BLK, H), lambda i: (i, 0)),
        out_shape=jax.ShapeDtypeStruct((N, H), jnp.float32),
    )(a1a, a1b, a2a, a2b, h, wihT, whhT, bih, bhh)


def _tc_last_idx(batch3):
    def body(b_ref, cnt_ref, last_ref):
        i = pl.program_id(0)

        @pl.when(i == 0)
        def _():
            cnt_ref[...] = jnp.zeros_like(cnt_ref)
            last_ref[...] = jnp.zeros_like(last_ref)

        bcol = b_ref[0, 0, :].reshape(BLK, 1)
        iota_b = lax.broadcasted_iota(jnp.int32, (BLK, B), 1)
        oh = jnp.where(bcol == iota_b, 1.0, 0.0)
        cnt_ref[...] += jnp.sum(oh, axis=0, keepdims=True)

        @pl.when(i == NBLK - 1)
        def _():
            cnt = cnt_ref[...]
            ir = lax.broadcasted_iota(jnp.int32, (B, B), 0)
            ic = lax.broadcasted_iota(jnp.int32, (B, B), 1)
            tri = jnp.where(ir <= ic, 1.0, 0.0)
            csum = jnp.dot(cnt, tri, preferred_element_type=jnp.float32)
            ci = csum.astype(jnp.int32)
            last_ref[...] = jnp.where(cnt > 0.0, ci - 1, 0)

    return pl.pallas_call(
        body,
        grid=(NBLK,),
        in_specs=[pl.BlockSpec((1, 1, BLK), lambda i: (i, 0, 0))],
        out_specs=[pl.BlockSpec((1, B), lambda i: (0, 0)),
                   pl.BlockSpec((1, B), lambda i: (0, 0))],
        out_shape=[jax.ShapeDtypeStruct((1, B), jnp.float32),
                   jax.ShapeDtypeStruct((1, B), jnp.int32)],
    )(batch3)


def _tc_alpha_sg(hidden, batch3, nc3, v_n, w1T, w2T, b12, qw, qb):
    def body(h_ref, b_ref, n_ref, vn_ref, w1_ref, w2_ref, b12_ref, qw_ref,
             qb_ref, sg_ref):
        i = pl.program_id(0)

        @pl.when(i == 0)
        def _():
            sg_ref[...] = jnp.zeros_like(sg_ref)

        hh = h_ref[...]
        bcol = b_ref[0, 0, :].reshape(BLK, 1)
        iota_b = lax.broadcasted_iota(jnp.int32, (BLK, B), 1)
        oh = jnp.where(bcol == iota_b, 1.0, 0.0)
        vrep = jnp.dot(oh, vn_ref[...], preferred_element_type=jnp.float32)
        t = jax.nn.sigmoid(
            jnp.dot(vrep, w1_ref[...], preferred_element_type=jnp.float32)
            + jnp.dot(hh, w2_ref[...], preferred_element_type=jnp.float32)
            + b12_ref[...])
        alpha = jnp.sum(t * qw_ref[...], axis=1, keepdims=True) + qb_ref[...]
        s = n_ref[0, 0, :].reshape(BLK, 1) * alpha * hh
        sg_ref[...] += lax.dot_general(oh, s, (((0,), (0,)), ((), ())),
                                       preferred_element_type=jnp.float32)

    return pl.pallas_call(
        body,
        grid=(NBLK,),
        in_specs=[
            pl.BlockSpec((BLK, H), lambda i: (i, 0)),
            pl.BlockSpec((1, 1, BLK), lambda i: (i, 0, 0)),
            pl.BlockSpec((1, 1, BLK), lambda i: (i, 0, 0)),
            pl.BlockSpec((B, H), lambda i: (0, 0)),
            pl.BlockSpec((H, H), lambda i: (0, 0)),
            pl.BlockSpec((H, H), lambda i: (0, 0)),
            pl.BlockSpec((1, H), lambda i: (0, 0)),
            pl.BlockSpec((1, H), lambda i: (0, 0)),
            pl.BlockSpec((1, 1), lambda i: (0, 0)),
        ],
        out_specs=pl.BlockSpec((B, H), lambda i: (0, 0)),
        out_shape=jax.ShapeDtypeStruct((B, H), jnp.float32),
    )(hidden, batch3, nc3, v_n, w1T, w2T, b12, qw, qb)


def _tc_logits(v_n, s_g, w3T, b3, emb):
    def body(vn_ref, sg_ref, w3_ref, b3_ref, emb_ref, out_ref):
        w3 = w3_ref[...]
        sh = (jnp.dot(vn_ref[...], w3[:H], preferred_element_type=jnp.float32)
              + jnp.dot(sg_ref[...], w3[H:], preferred_element_type=jnp.float32)
              + b3_ref[...])
        out_ref[...] = lax.dot_general(sh, emb_ref[...],
                                       (((1,), (1,)), ((), ())),
                                       preferred_element_type=jnp.float32)

    return pl.pallas_call(
        body,
        grid=(NVBLK,),
        in_specs=[
            pl.BlockSpec((B, H), lambda i: (0, 0)),
            pl.BlockSpec((B, H), lambda i: (0, 0)),
            pl.BlockSpec((2 * H, H), lambda i: (0, 0)),
            pl.BlockSpec((1, H), lambda i: (0, 0)),
            pl.BlockSpec((VBLK, H), lambda i: (i, 0)),
        ],
        out_specs=pl.BlockSpec((B, VBLK), lambda i: (0, i)),
        out_shape=jax.ShapeDtypeStruct((B, NNP), jnp.float32),
    )(v_n, s_g, w3T, b3, emb)


def kernel(x, edge_index, batch, edge_count, in_degree_inv, out_degree_inv,
           num_count, sequence, emb_table, weight_in, weight_out, gru_w_ih,
           gru_w_hh, gru_b_ih, gru_b_hh, W1_w, W1_b, W2_w, W2_b, q_w, q_b,
           W3_w, W3_b):
    xi = x.reshape(-1).astype(jnp.int32)
    xp2 = jnp.concatenate([xi, jnp.ones((NP - N,), jnp.int32)]).reshape(
        NW, G_BURSTS, 128)
    h = _sc_emb_gather(emb_table, xp2)[:N]

    m1a, m1b, m2a, m2b = _tc_m1m2(h, weight_in[0], weight_out[0])

    pad = EP - E
    zi = jnp.zeros((pad,), jnp.int32)
    zf = jnp.zeros((pad,), jnp.float32)
    src2 = jnp.concatenate([edge_index[0].astype(jnp.int32), zi]).reshape(-1, 128)
    dst2 = jnp.concatenate([edge_index[1].astype(jnp.int32), zi]).reshape(-1, 128)
    ec2 = jnp.concatenate([edge_count, zf]).reshape(-1, 128)
    din2 = jnp.concatenate([in_degree_inv, zf]).reshape(-1, 128)
    dout2 = jnp.concatenate([out_degree_inv, zf]).reshape(-1, 128)
    a1a, a1b, a2a, a2b = _sc_edge_agg(m1a, m1b, m2a, m2b, src2, dst2, ec2,
                                      din2, dout2)

    hidden = _tc_gru(a1a, a1b, a2a, a2b, h, gru_w_ih.T, gru_w_hh.T,
                     gru_b_ih.reshape(1, -1), gru_b_hh.reshape(1, -1))

    batch3 = batch.astype(jnp.int32).reshape(NBLK, 1, BLK)
    _, last_idx = _tc_last_idx(batch3)
    v_n = _sc_vn_gather(hidden, last_idx.reshape(-1))

    nc3 = num_count.reshape(NBLK, 1, BLK)
    s_g = _tc_alpha_sg(hidden, batch3, nc3, v_n, W1_w.T, W2_w.T,
                       (W1_b + W2_b).reshape(1, -1), q_w.reshape(1, -1),
                       q_b.reshape(1, 1))

    embp = jnp.pad(emb_table, ((0, NNP - N_NODE), (0, 0)))
    z = _tc_logits(v_n, s_g, W3_w.T, W3_b.reshape(1, -1), embp)
    return z[:, :N_NODE]


# speedup vs baseline: 4.3787x; 1.0001x over previous
import functools

import jax
import jax.numpy as jnp
from jax import lax
from jax.experimental import pallas as pl
from jax.experimental.pallas import tpu as pltpu
from jax.experimental.pallas import tpu_sc as plsc

N = 50000
H = 64
B = 256
E = 800000
N_NODE = 100000

NC = 2
NS = 16
NW = NC * NS

GP_ROWS = 128
G_BURSTS = 13
ROWS_W = GP_ROWS * G_BURSTS
NP = ROWS_W * NW

EP = 819200
ET = EP // NS
CH = 128
SB = 8
RING = 4
NSUP = ET // (CH * SB)
HH = H // 2
ZR = N // NS

BLK = 2000
NBLK = N // BLK
EMB_R = 80
EMB_C = 1250
EMB_B = 8


def _mesh():
    return plsc.VectorSubcoreMesh(core_axis_name="c", subcore_axis_name="s")


def _sc_emb_gather(table, idx2):
    @functools.partial(
        pl.kernel,
        out_type=(jax.ShapeDtypeStruct((NP, HH), jnp.float32),
                  jax.ShapeDtypeStruct((NP, HH), jnp.float32)),
        mesh=_mesh(),
        compiler_params=pltpu.CompilerParams(use_tc_tiling_on_sc=False),
        scratch_types=[
            pltpu.VMEM((G_BURSTS, GP_ROWS), jnp.int32),
            pltpu.VMEM((ROWS_W, H), jnp.float32),
            pltpu.SemaphoreType.DMA,
        ],
    )
    def k(table_h, idx_h, outa_h, outb_h, idx_v, rows_v, sem):
        w = lax.axis_index("s") * NC + lax.axis_index("c")
        pltpu.sync_copy(idx_h.at[w], idx_v)
        for r in range(G_BURSTS):
            def sub1(i, _, r=r):
                sl = pl.ds(i * 16, 16)
                idx_v[r, sl] = idx_v[r, sl] - 1
                return 0
            lax.fori_loop(0, GP_ROWS // 16, sub1, 0)
        cps = [
            pltpu.async_copy(table_h.at[idx_v.at[r]],
                             rows_v.at[pl.ds(r * GP_ROWS, GP_ROWS)], sem)
            for r in range(G_BURSTS)
        ]
        for cp in cps:
            cp.wait()
        pltpu.sync_copy(rows_v.at[:, pl.ds(0, HH)],
                        outa_h.at[pl.ds(w * ROWS_W, ROWS_W)])
        pltpu.sync_copy(rows_v.at[:, pl.ds(HH, HH)],
                        outb_h.at[pl.ds(w * ROWS_W, ROWS_W)])

    return k(table, idx2)


def _sc_vn_gather(hidden, last_idx):
    rw = B // NW

    @functools.partial(
        pl.kernel,
        out_type=jax.ShapeDtypeStruct((B, H), jnp.float32),
        mesh=_mesh(),
        compiler_params=pltpu.CompilerParams(use_tc_tiling_on_sc=False),
        scratch_types=[
            pltpu.VMEM((rw,), jnp.int32),
            pltpu.VMEM((rw, H), jnp.float32),
            pltpu.SemaphoreType.DMA,
        ],
    )
    def k(hid_h, idx_h, out_h, idx_v, rows_v, sem):
        w = lax.axis_index("s") * NC + lax.axis_index("c")
        pltpu.sync_copy(idx_h.at[pl.ds(w * rw, rw)], idx_v)
        pltpu.async_copy(hid_h.at[idx_v], rows_v, sem).wait()
        pltpu.sync_copy(rows_v, out_h.at[pl.ds(w * rw, rw)])

    return k(hidden, last_idx)


def _sc_edge_agg(ha, hb, src2, dst2, ec2, din2, dout2):
    out_t = tuple(jax.ShapeDtypeStruct((N, HH), jnp.float32) for _ in range(4))

    @functools.partial(
        pl.kernel,
        out_type=out_t,
        mesh=_mesh(),
        compiler_params=pltpu.CompilerParams(use_tc_tiling_on_sc=False),
        scratch_types=[
            pltpu.VMEM_SHARED((N, HH), jnp.float32),
            pltpu.VMEM((SB, 128), jnp.int32),
            pltpu.VMEM((SB, 128), jnp.int32),
            pltpu.VMEM((SB, 128), jnp.float32),
            pltpu.VMEM((SB, 128), jnp.float32),
            pltpu.VMEM((RING * CH, HH), jnp.float32),
            pltpu.SemaphoreType.DMA,
            pltpu.SemaphoreType.DMA,
        ],
    )
    def k(ha_h, hb_h, src_h, dst_h, ec_h, din_h, dout_h,
          o1a_h, o1b_h, o2a_h, o2b_h,
          acc, gidx_v, tidx_v, ec_v, ew_v, rows_v, semi, semg):
        c = lax.axis_index("c")
        s = lax.axis_index("s")

        def one_direction(mat_h, g_h, t_h, w_h, out_h):
            def zrow(kk, _):
                zz = jnp.zeros((16,), jnp.float32)
                for q in range(HH // 16):
                    rows_v[kk, pl.ds(q * 16, 16)] = zz
                return 0
            lax.fori_loop(0, RING * CH, zrow, 0)
            zb = RING * CH
            for off in range(0, ZR - zb + 1, zb):
                pltpu.sync_copy(rows_v, acc.at[pl.ds(s * ZR + off, zb)])
            rem = ZR % zb
            if rem:
                pltpu.sync_copy(rows_v.at[pl.ds(0, rem)],
                                acc.at[pl.ds(s * ZR + ZR - rem, rem)])
            plsc.subcore_barrier()

            def super_chunk(sj, _):
                rb = s * (ET // 128) + sj * SB
                cp_i = [pltpu.async_copy(g_h.at[pl.ds(rb, SB)], gidx_v, semi),
                        pltpu.async_copy(t_h.at[pl.ds(rb, SB)], tidx_v, semi),
                        pltpu.async_copy(ec_h.at[pl.ds(rb, SB)], ec_v, semi),
                        pltpu.async_copy(w_h.at[pl.ds(rb, SB)], ew_v, semi)]
                for cp in cp_i:
                    cp.wait()

                def prep_r(r, _):
                    def prep_i(i, _):
                        sl = pl.ds(i * 16, 16)
                        ew_v[r, sl] = ec_v[r, sl] * ew_v[r, sl]
                        return 0
                    lax.fori_loop(0, 8, prep_i, 0)
                    return 0
                lax.fori_loop(0, SB, prep_r, 0)

                cps = [None] * RING
                for p in range(RING - 1):
                    cps[p] = pltpu.async_copy(
                        mat_h.at[gidx_v.at[p]],
                        rows_v.at[pl.ds(p * CH, CH)], semg)
                for kc in range(SB):
                    b = kc % RING
                    nx = kc + RING - 1
                    if nx < SB:
                        cps[nx % RING] = pltpu.async_copy(
                            mat_h.at[gidx_v.at[nx]],
                            rows_v.at[pl.ds((nx % RING) * CH, CH)], semg)
                    cps[b].wait()

                    def scale_b(kb, _, kc=kc, b=b):
                        w16 = ew_v[kc, pl.ds(kb * 16, 16)]
                        base = b * CH + kb * 16
                        for kk in range(16):
                            wk = w16[kk]
                            for q in range(HH // 16):
                                sl = pl.ds(q * 16, 16)
                                rows_v[base + kk, sl] = rows_v[base + kk, sl] * wk
                        return 0
                    lax.fori_loop(0, CH // 16, scale_b, 0)

                    pltpu.sync_copy(rows_v.at[pl.ds(b * CH, CH)],
                                    acc.at[tidx_v.at[kc]], add=True)
                return 0
            lax.fori_loop(0, NSUP, super_chunk, 0)
            plsc.subcore_barrier()

            pltpu.sync_copy(acc.at[pl.ds(s * ZR, ZR)],
                            out_h.at[pl.ds(s * ZR, ZR)])
            plsc.subcore_barrier()

        @pl.when(c == 0)
        def _():
            one_direction(ha_h, src_h, dst_h, din_h, o1a_h)
            one_direction(ha_h, dst_h, src_h, dout_h, o2a_h)

        @pl.when(c == 1)
        def _():
            one_direction(hb_h, src_h, dst_h, din_h, o1b_h)
            one_direction(hb_h, dst_h, src_h, dout_h, o2b_h)

    return k(ha, hb, src2, dst2, ec2, din2, dout2)


def _tc_gru(p1a, p1b, p2a, p2b, ha, hb, win, wout, wihT, whhT, bih, bhh):
    def body(p1a_ref, p1b_ref, p2a_ref, p2b_ref, ha_ref, hb_ref, wi_ref, wo_ref,
             wih_ref, whh_ref, bih_ref, bhh_ref, out_ref):
        hh = jnp.concatenate([ha_ref[...], hb_ref[...]], axis=1)
        wih = wih_ref[...]
        ww1 = jnp.dot(wi_ref[...], wih[:H], preferred_element_type=jnp.float32)
        ww2 = jnp.dot(wo_ref[...], wih[H:], preferred_element_type=jnp.float32)
        gi = (jnp.dot(p1a_ref[...], ww1[:HH], preferred_element_type=jnp.float32)
              + jnp.dot(p1b_ref[...], ww1[HH:], preferred_element_type=jnp.float32)
              + jnp.dot(p2a_ref[...], ww2[:HH], preferred_element_type=jnp.float32)
              + jnp.dot(p2b_ref[...], ww2[HH:], preferred_element_type=jnp.float32)
              + bih_ref[...])
        gh = jnp.dot(hh, whh_ref[...], preferred_element_type=jnp.float32) + bhh_ref[...]
        r = jax.nn.sigmoid(gi[:, :H] + gh[:, :H])
        z = jax.nn.sigmoid(gi[:, H:2 * H] + gh[:, H:2 * H])
        ng = jnp.tanh(gi[:, 2 * H:] + r * gh[:, 2 * H:])
        out_ref[...] = (1.0 - z) * ng + z * hh

    return pl.pallas_call(
        body,
        grid=(NBLK,),
        in_specs=[
            pl.BlockSpec((BLK, HH), lambda i: (i, 0)),
            pl.BlockSpec((BLK, HH), lambda i: (i, 0)),
            pl.BlockSpec((BLK, HH), lambda i: (i, 0)),
            pl.BlockSpec((BLK, HH), lambda i: (i, 0)),
            pl.BlockSpec((BLK, HH), lambda i: (i, 0)),
            pl.BlockSpec((BLK, HH), lambda i: (i, 0)),
            pl.BlockSpec((H, H), lambda i: (0, 0)),
            pl.BlockSpec((H, H), lambda i: (0, 0)),
            pl.BlockSpec((2 * H, 3 * H), lambda i: (0, 0)),
            pl.BlockSpec((H, 3 * H), lambda i: (0, 0)),
            pl.BlockSpec((1, 3 * H), lambda i: (0, 0)),
            pl.BlockSpec((1, 3 * H), lambda i: (0, 0)),
        ],
        out_specs=pl.BlockSpec((BLK, H), lambda i: (i, 0)),
        out_shape=jax.ShapeDtypeStruct((N, H), jnp.float32),
    )(p1a, p1b, p2a, p2b, ha, hb, win, wout, wihT, whhT, bih, bhh)


def _tc_last_idx(batch3):
    def body(b_ref, cnt_ref, last_ref):
        i = pl.program_id(0)

        @pl.when(i == 0)
        def _():
            cnt_ref[...] = jnp.zeros_like(cnt_ref)
            last_ref[...] = jnp.zeros_like(last_ref)

        bcol = b_ref[0, 0, :].reshape(BLK, 1)
        iota_b = lax.broadcasted_iota(jnp.int32, (BLK, B), 1)
        oh = jnp.where(bcol == iota_b, 1.0, 0.0)
        cnt_ref[...] += jnp.sum(oh, axis=0, keepdims=True)

        @pl.when(i == NBLK - 1)
        def _():
            cnt = cnt_ref[...]
            ir = lax.broadcasted_iota(jnp.int32, (B, B), 0)
            ic = lax.broadcasted_iota(jnp.int32, (B, B), 1)
            tri = jnp.where(ir <= ic, 1.0, 0.0)
            csum = jnp.dot(cnt, tri, preferred_element_type=jnp.float32)
            ci = csum.astype(jnp.int32)
            last_ref[...] = jnp.where(cnt > 0.0, ci - 1, 0)

    return pl.pallas_call(
        body,
        grid=(NBLK,),
        in_specs=[pl.BlockSpec((1, 1, BLK), lambda i: (i, 0, 0))],
        out_specs=[pl.BlockSpec((1, B), lambda i: (0, 0)),
                   pl.BlockSpec((1, B), lambda i: (0, 0))],
        out_shape=[jax.ShapeDtypeStruct((1, B), jnp.float32),
                   jax.ShapeDtypeStruct((1, B), jnp.int32)],
    )(batch3)


def _tc_alpha_sg(hidden, batch3, nc3, v_n, w1T, w2T, b12, qw, qb):
    def body(h_ref, b_ref, n_ref, vn_ref, w1_ref, w2_ref, b12_ref, qw_ref,
             qb_ref, sg_ref):
        i = pl.program_id(0)

        @pl.when(i == 0)
        def _():
            sg_ref[...] = jnp.zeros_like(sg_ref)

        hh = h_ref[...]
        bcol = b_ref[0, 0, :].reshape(BLK, 1)
        iota_b = lax.broadcasted_iota(jnp.int32, (BLK, B), 1)
        oh = jnp.where(bcol == iota_b, 1.0, 0.0)
        vrep = jnp.dot(oh, vn_ref[...], preferred_element_type=jnp.float32)
        t = jax.nn.sigmoid(
            jnp.dot(vrep, w1_ref[...], preferred_element_type=jnp.float32)
            + jnp.dot(hh, w2_ref[...], preferred_element_type=jnp.float32)
            + b12_ref[...])
        alpha = jnp.sum(t * qw_ref[...], axis=1, keepdims=True) + qb_ref[...]
        s = n_ref[0, 0, :].reshape(BLK, 1) * alpha * hh
        sg_ref[...] += lax.dot_general(oh, s, (((0,), (0,)), ((), ())),
                                       preferred_element_type=jnp.float32)

    return pl.pallas_call(
        body,
        grid=(NBLK,),
        in_specs=[
            pl.BlockSpec((BLK, H), lambda i: (i, 0)),
            pl.BlockSpec((1, 1, BLK), lambda i: (i, 0, 0)),
            pl.BlockSpec((1, 1, BLK), lambda i: (i, 0, 0)),
            pl.BlockSpec((B, H), lambda i: (0, 0)),
            pl.BlockSpec((H, H), lambda i: (0, 0)),
            pl.BlockSpec((H, H), lambda i: (0, 0)),
            pl.BlockSpec((1, H), lambda i: (0, 0)),
            pl.BlockSpec((1, H), lambda i: (0, 0)),
            pl.BlockSpec((1, 1), lambda i: (0, 0)),
        ],
        out_specs=pl.BlockSpec((B, H), lambda i: (0, 0)),
        out_shape=jax.ShapeDtypeStruct((B, H), jnp.float32),
    )(hidden, batch3, nc3, v_n, w1T, w2T, b12, qw, qb)


def _tc_logits(v_n, s_g, w3T, b3, emb3):
    def body(vn_ref, sg_ref, w3_ref, b3_ref, emb_ref, out_ref):
        w3 = w3_ref[...]
        sh = (jnp.dot(vn_ref[...], w3[:H], preferred_element_type=jnp.float32)
              + jnp.dot(sg_ref[...], w3[H:], preferred_element_type=jnp.float32)
              + b3_ref[...])
        er = emb_ref[...].reshape(EMB_B * EMB_C, H)
        zz = lax.dot_general(sh, er, (((1,), (1,)), ((), ())),
                             preferred_element_type=jnp.float32)
        out_ref[...] = zz.reshape(B, EMB_B, EMB_C)

    return pl.pallas_call(
        body,
        grid=(EMB_R // EMB_B,),
        in_specs=[
            pl.BlockSpec((B, H), lambda i: (0, 0)),
            pl.BlockSpec((B, H), lambda i: (0, 0)),
            pl.BlockSpec((2 * H, H), lambda i: (0, 0)),
            pl.BlockSpec((1, H), lambda i: (0, 0)),
            pl.BlockSpec((EMB_B, EMB_C, H), lambda i: (i, 0, 0)),
        ],
        out_specs=pl.BlockSpec((B, EMB_B, EMB_C), lambda i: (0, i, 0)),
        out_shape=jax.ShapeDtypeStruct((B, EMB_R, EMB_C), jnp.float32),
    )(v_n, s_g, w3T, b3, emb3)


def kernel(x, edge_index, batch, edge_count, in_degree_inv, out_degree_inv,
           num_count, sequence, emb_table, weight_in, weight_out, gru_w_ih,
           gru_w_hh, gru_b_ih, gru_b_hh, W1_w, W1_b, W2_w, W2_b, q_w, q_b,
           W3_w, W3_b):
    xi = x.reshape(-1).astype(jnp.int32)
    xp2 = jnp.concatenate([xi, jnp.ones((NP - N,), jnp.int32)]).reshape(
        NW, G_BURSTS, 128)
    ha, hb = _sc_emb_gather(emb_table, xp2)

    pad = EP - E
    zi = jnp.zeros((pad,), jnp.int32)
    zf = jnp.zeros((pad,), jnp.float32)
    src2 = jnp.concatenate([edge_index[0].astype(jnp.int32), zi]).reshape(-1, 128)
    dst2 = jnp.concatenate([edge_index[1].astype(jnp.int32), zi]).reshape(-1, 128)
    ec2 = jnp.concatenate([edge_count, zf]).reshape(-1, 128)
    din2 = jnp.concatenate([in_degree_inv, zf]).reshape(-1, 128)
    dout2 = jnp.concatenate([out_degree_inv, zf]).reshape(-1, 128)
    a1a, a1b, a2a, a2b = _sc_edge_agg(ha, hb, src2, dst2, ec2, din2, dout2)

    hidden = _tc_gru(a1a, a1b, a2a, a2b, ha[:N], hb[:N],
                     weight_in[0], weight_out[0], gru_w_ih.T, gru_w_hh.T,
                     gru_b_ih.reshape(1, -1), gru_b_hh.reshape(1, -1))

    batch3 = batch.astype(jnp.int32).reshape(NBLK, 1, BLK)
    _, last_idx = _tc_last_idx(batch3)
    v_n = _sc_vn_gather(hidden, last_idx.reshape(-1))

    nc3 = num_count.reshape(NBLK, 1, BLK)
    s_g = _tc_alpha_sg(hidden, batch3, nc3, v_n, W1_w.T, W2_w.T,
                       (W1_b + W2_b).reshape(1, -1), q_w.reshape(1, -1),
                       q_b.reshape(1, 1))

    emb3 = emb_table.reshape(EMB_R, EMB_C, H)
    z = _tc_logits(v_n, s_g, W3_w.T, W3_b.reshape(1, -1), emb3)
    return z.reshape(B, N_NODE)
